# baseline (speedup 1.0000x reference)
import functools

import jax
import jax.numpy as jnp
from jax import lax
from jax.experimental import pallas as pl
from jax.experimental.pallas import tpu as pltpu
from jax.experimental.pallas import tpu_sc as plsc

_SC_NC, _SC_NS, _SC_L = 2, 16, 16


def _sc_build_a_kernel(idx_hbm, w_hbm, a_hbm, idx_v, w_v, acc_v, *,
                       rows_per_worker, K, E):
    wid = lax.axis_index("s") * _SC_NC + lax.axis_index("c")
    base_row = wid * rows_per_worker
    n_pairs = rows_per_worker * K
    pltpu.sync_copy(idx_hbm.at[pl.ds(base_row * K, n_pairs)], idx_v)
    pltpu.sync_copy(w_hbm.at[pl.ds(base_row * K, n_pairs)], w_v)

    zeros16 = jnp.zeros((_SC_L,), jnp.float32)

    def _zero(i, c):
        acc_v[pl.ds(i * _SC_L, _SC_L)] = zeros16
        return c

    lax.fori_loop(0, (rows_per_worker * E) // _SC_L, _zero, 0)

    lane = lax.iota(jnp.int32, _SC_L)

    def _scatter(i, c):
        rows = i * _SC_L + lane
        for k in range(K):
            iv = plsc.load_gather(idx_v, [rows * K + k])
            wv = plsc.load_gather(w_v, [rows * K + k])
            plsc.addupdate_scatter(acc_v, [rows * E + iv], wv)
        return c

    lax.fori_loop(0, rows_per_worker // _SC_L, _scatter, 0)

    pltpu.sync_copy(acc_v, a_hbm.at[pl.ds(base_row * E, rows_per_worker * E)])


def _sc_build_a(idx_flat, w_flat, R, K, E):
    rows_per_worker = R // (_SC_NC * _SC_NS)
    n_pairs = rows_per_worker * K
    kfn = pl.kernel(
        functools.partial(_sc_build_a_kernel, rows_per_worker=rows_per_worker,
                          K=K, E=E),
        mesh=plsc.VectorSubcoreMesh(core_axis_name="c", subcore_axis_name="s"),
        compiler_params=pltpu.CompilerParams(needs_layout_passes=False),
        out_type=jax.ShapeDtypeStruct((R * E,), jnp.float32),
        scratch_types=[
            pltpu.VMEM((n_pairs,), jnp.int32),
            pltpu.VMEM((n_pairs,), jnp.float32),
            pltpu.VMEM((rows_per_worker * E,), jnp.float32),
        ],
    )
    return kfn(idx_flat, w_flat)


def _gelu_exact(x):
    return x * 0.5 * (1.0 + jax.lax.erf(x * 0.7071067811865476))


def _ln(x, g, b, eps=1e-5):
    mu = jnp.mean(x, axis=-1, keepdims=True)
    var = jnp.mean((x - mu) ** 2, axis=-1, keepdims=True)
    return (x - mu) * jax.lax.rsqrt(var + eps) * g + b


def _fused_kernel(a_ref, x_ref, We_ref, be_ref, ge_ref, bbe_ref, Wn_ref,
                  bn_ref, gn_ref, bbn_ref, out_ref, he_s, he2_s, xbf_s, *,
                  NB, Tn):
    nb = pl.program_id(1)

    @pl.when(nb < NB)
    def _phase1():
        x = x_ref[0]
        xbf_s[pl.ds(nb * Tn, Tn), :] = x.astype(jnp.bfloat16)
        a = a_ref[0]
        acc = jax.lax.dot_general(a, x, (((0,), (0,)), ((), ())),
                                  preferred_element_type=jnp.float32)

        @pl.when(nb == 0)
        def _():
            he_s[...] = acc

        @pl.when(nb != 0)
        def _():
            he_s[...] += acc

    @pl.when(nb == NB)
    def _phase2():
        y = jax.lax.dot_general(he_s[...], We_ref[...],
                                (((1,), (0,)), ((), ())),
                                preferred_element_type=jnp.float32)
        y = _gelu_exact(y + be_ref[...])
        he2_s[...] = _ln(y, ge_ref[...], bbe_ref[...])

    @pl.when(nb > NB)
    def _phase3():
        j = nb - NB - 1
        a = a_ref[0]
        xg = jax.lax.dot_general(a, he2_s[...], (((1,), (0,)), ((), ())),
                                 preferred_element_type=jnp.float32)
        y = jax.lax.dot_general(xg, Wn_ref[...], (((1,), (0,)), ((), ())),
                                preferred_element_type=jnp.float32)
        y = _gelu_exact(y + bn_ref[...])
        y = _ln(y, gn_ref[...], bbn_ref[...])
        out_ref[0] = y + xbf_s[pl.ds(j * Tn, Tn), :].astype(jnp.float32)


def kernel(X, edge_idx, edge_w, We, be, ge, bbe, Wn, bn, gn, bbn):
    B, N, D = X.shape
    K = edge_idx.shape[-1]
    E = 64
    Tn = 2048
    NB = N // Tn
    S = 2 * NB + 1

    idx = edge_idx.astype(jnp.int32)

    a_mat = _sc_build_a(idx.reshape(B * N * K), edge_w.reshape(B * N * K),
                        B * N, K, E).reshape(B, N, E)

    be2 = be.reshape(1, D)
    ge2 = ge.reshape(1, D)
    bbe2 = bbe.reshape(1, D)
    bn2 = bn.reshape(1, D)
    gn2 = gn.reshape(1, D)
    bbn2 = bbn.reshape(1, D)

    def a_map(b, nb):
        j = jnp.where(nb < NB, nb,
                      jnp.clip(nb - NB - 1, 0, NB - 1))
        return (b, j, 0)

    def x_map(b, nb):
        return (b, jnp.minimum(nb, NB - 1), 0)

    def out_map(b, nb):
        return (b, jnp.clip(nb - NB - 1, 0, NB - 1), 0)

    const = lambda b, nb: (0, 0)

    out = pl.pallas_call(
        functools.partial(_fused_kernel, NB=NB, Tn=Tn),
        grid=(B, S),
        in_specs=[
            pl.BlockSpec((1, Tn, E), a_map),
            pl.BlockSpec((1, Tn, D), x_map),
            pl.BlockSpec(We.shape, const),
            pl.BlockSpec((1, D), const),
            pl.BlockSpec((1, D), const),
            pl.BlockSpec((1, D), const),
            pl.BlockSpec(Wn.shape, const),
            pl.BlockSpec((1, D), const),
            pl.BlockSpec((1, D), const),
            pl.BlockSpec((1, D), const),
        ],
        out_specs=pl.BlockSpec((1, Tn, D), out_map),
        out_shape=jax.ShapeDtypeStruct((B, N, D), jnp.float32),
        compiler_params=pltpu.CompilerParams(
            vmem_limit_bytes=100 * 1024 * 1024),
        scratch_shapes=[
            pltpu.VMEM((E, D), jnp.float32),
            pltpu.VMEM((E, D), jnp.float32),
            pltpu.VMEM((N, D), jnp.bfloat16),
        ],
    )(a_mat, X, We, be2, ge2, bbe2, Wn, bn2, gn2, bbn2)

    return out


# speedup vs baseline: 1.0103x; 1.0103x over previous
import functools

import jax
import jax.numpy as jnp
from jax import lax
from jax.experimental import pallas as pl
from jax.experimental.pallas import tpu as pltpu
from jax.experimental.pallas import tpu_sc as plsc

_SC_NC, _SC_NS, _SC_L = 2, 16, 16


def _sc_build_a_kernel(idx_hbm, w_hbm, a_hbm, idx_v, w_v, acc_v, *,
                       rows_per_worker, K, E):
    wid = lax.axis_index("s") * _SC_NC + lax.axis_index("c")
    base_row = wid * rows_per_worker
    n_pairs = rows_per_worker * K
    pltpu.sync_copy(idx_hbm.at[pl.ds(base_row * K, n_pairs)], idx_v)
    pltpu.sync_copy(w_hbm.at[pl.ds(base_row * K, n_pairs)], w_v)

    zeros16 = jnp.zeros((_SC_L,), jnp.float32)

    def _zero(i, c):
        acc_v[pl.ds(i * _SC_L, _SC_L)] = zeros16
        return c

    lax.fori_loop(0, (rows_per_worker * E) // _SC_L, _zero, 0)

    lane = lax.iota(jnp.int32, _SC_L)

    def _scatter(i, c):
        rows = i * _SC_L + lane
        for k in range(K):
            iv = plsc.load_gather(idx_v, [rows * K + k])
            wv = plsc.load_gather(w_v, [rows * K + k])
            plsc.addupdate_scatter(acc_v, [rows * E + iv], wv)
        return c

    lax.fori_loop(0, rows_per_worker // _SC_L, _scatter, 0)

    pltpu.sync_copy(acc_v, a_hbm.at[pl.ds(base_row * E, rows_per_worker * E)])


def _sc_build_a(idx_flat, w_flat, R, K, E):
    rows_per_worker = R // (_SC_NC * _SC_NS)
    n_pairs = rows_per_worker * K
    kfn = pl.kernel(
        functools.partial(_sc_build_a_kernel, rows_per_worker=rows_per_worker,
                          K=K, E=E),
        mesh=plsc.VectorSubcoreMesh(core_axis_name="c", subcore_axis_name="s"),
        compiler_params=pltpu.CompilerParams(needs_layout_passes=False),
        out_type=jax.ShapeDtypeStruct((R * E,), jnp.float32),
        scratch_types=[
            pltpu.VMEM((n_pairs,), jnp.int32),
            pltpu.VMEM((n_pairs,), jnp.float32),
            pltpu.VMEM((rows_per_worker * E,), jnp.float32),
        ],
    )
    return kfn(idx_flat, w_flat)


def _gelu_exact(x):
    return x * 0.5 * (1.0 + jax.lax.erf(x * 0.7071067811865476))


def _ln(x, g, b, eps=1e-5):
    mu = jnp.mean(x, axis=-1, keepdims=True)
    var = jnp.mean((x - mu) ** 2, axis=-1, keepdims=True)
    return (x - mu) * jax.lax.rsqrt(var + eps) * g + b


def _fused_kernel(a_ref, x_ref, We_ref, be_ref, ge_ref, bbe_ref, Wn_ref,
                  bn_ref, gn_ref, bbn_ref, out_ref, he_s, he2_s, xbf_s, *,
                  NB, Tn):
    nb = pl.program_id(1)

    @pl.when(nb < NB)
    def _phase1():
        x = x_ref[0]
        xbf_s[pl.ds(nb * Tn, Tn), :] = x.astype(jnp.bfloat16)
        a = a_ref[0]
        acc = jax.lax.dot_general(a, x, (((0,), (0,)), ((), ())),
                                  preferred_element_type=jnp.float32)

        @pl.when(nb == 0)
        def _():
            he_s[...] = acc

        @pl.when(nb != 0)
        def _():
            he_s[...] += acc

    @pl.when(nb == NB)
    def _phase2():
        y = jax.lax.dot_general(he_s[...], We_ref[...],
                                (((1,), (0,)), ((), ())),
                                preferred_element_type=jnp.float32)
        y = _gelu_exact(y + be_ref[...])
        he2_s[...] = _ln(y, ge_ref[...], bbe_ref[...])

    @pl.when(nb > NB)
    def _phase3():
        j = nb - NB - 1
        a = a_ref[0]
        xg = jax.lax.dot_general(a, he2_s[...], (((1,), (0,)), ((), ())),
                                 preferred_element_type=jnp.float32)
        y = jax.lax.dot_general(xg, Wn_ref[...], (((1,), (0,)), ((), ())),
                                preferred_element_type=jnp.float32)
        y = _gelu_exact(y + bn_ref[...])
        y = _ln(y, gn_ref[...], bbn_ref[...])
        out_ref[0] = y + xbf_s[pl.ds(j * Tn, Tn), :].astype(jnp.float32)


def kernel(X, edge_idx, edge_w, We, be, ge, bbe, Wn, bn, gn, bbn):
    B, N, D = X.shape
    K = edge_idx.shape[-1]
    E = 64
    Tn = 1024
    NB = N // Tn
    S = 2 * NB + 1

    idx = edge_idx.astype(jnp.int32)

    a_mat = _sc_build_a(idx.reshape(B * N * K), edge_w.reshape(B * N * K),
                        B * N, K, E).reshape(B, N, E)

    be2 = be.reshape(1, D)
    ge2 = ge.reshape(1, D)
    bbe2 = bbe.reshape(1, D)
    bn2 = bn.reshape(1, D)
    gn2 = gn.reshape(1, D)
    bbn2 = bbn.reshape(1, D)

    def a_map(b, nb):
        j = jnp.where(nb < NB, nb,
                      jnp.clip(nb - NB - 1, 0, NB - 1))
        return (b, j, 0)

    def x_map(b, nb):
        return (b, jnp.minimum(nb, NB - 1), 0)

    def out_map(b, nb):
        return (b, jnp.clip(nb - NB - 1, 0, NB - 1), 0)

    const = lambda b, nb: (0, 0)

    out = pl.pallas_call(
        functools.partial(_fused_kernel, NB=NB, Tn=Tn),
        grid=(B, S),
        in_specs=[
            pl.BlockSpec((1, Tn, E), a_map),
            pl.BlockSpec((1, Tn, D), x_map),
            pl.BlockSpec(We.shape, const),
            pl.BlockSpec((1, D), const),
            pl.BlockSpec((1, D), const),
            pl.BlockSpec((1, D), const),
            pl.BlockSpec(Wn.shape, const),
            pl.BlockSpec((1, D), const),
            pl.BlockSpec((1, D), const),
            pl.BlockSpec((1, D), const),
        ],
        out_specs=pl.BlockSpec((1, Tn, D), out_map),
        out_shape=jax.ShapeDtypeStruct((B, N, D), jnp.float32),
        compiler_params=pltpu.CompilerParams(
            vmem_limit_bytes=100 * 1024 * 1024,
            fuse_transposed_lhs_in_matmul=True),
        scratch_shapes=[
            pltpu.VMEM((E, D), jnp.float32),
            pltpu.VMEM((E, D), jnp.float32),
            pltpu.VMEM((N, D), jnp.bfloat16),
        ],
    )(a_mat, X, We, be2, ge2, bbe2, Wn, bn2, gn2, bbn2)

    return out


# speedup vs baseline: 1.0663x; 1.0554x over previous
import functools

import jax
import jax.numpy as jnp
from jax import lax
from jax.experimental import pallas as pl
from jax.experimental.pallas import tpu as pltpu
from jax.experimental.pallas import tpu_sc as plsc

_SC_NC, _SC_NS, _SC_L = 2, 16, 16


def _sc_build_a_kernel(idx_hbm, w_hbm, a_hbm, idx_v, w_v, acc_v, *,
                       rows_per_worker, K, E):
    wid = lax.axis_index("s") * _SC_NC + lax.axis_index("c")
    base_row = wid * rows_per_worker
    n_pairs = rows_per_worker * K
    pltpu.sync_copy(idx_hbm.at[pl.ds(base_row * K, n_pairs)], idx_v)
    pltpu.sync_copy(w_hbm.at[pl.ds(base_row * K, n_pairs)], w_v)

    zeros16 = jnp.zeros((_SC_L,), jnp.float32)
    _ZU = 8

    def _zero(i, c):
        for u in range(_ZU):
            acc_v[pl.ds((i * _ZU + u) * _SC_L, _SC_L)] = zeros16
        return c

    lax.fori_loop(0, (rows_per_worker * E) // (_SC_L * _ZU), _zero, 0)

    lane = lax.iota(jnp.int32, _SC_L)

    def _scatter(i, c):
        rows = i * _SC_L + lane
        for k in range(K):
            iv = plsc.load_gather(idx_v, [rows * K + k])
            wv = plsc.load_gather(w_v, [rows * K + k])
            plsc.addupdate_scatter(acc_v, [rows * E + iv], wv)
        return c

    lax.fori_loop(0, rows_per_worker // _SC_L, _scatter, 0)

    pltpu.sync_copy(acc_v, a_hbm.at[pl.ds(base_row * E, rows_per_worker * E)])


def _sc_build_a(idx_flat, w_flat, R, K, E):
    rows_per_worker = R // (_SC_NC * _SC_NS)
    n_pairs = rows_per_worker * K
    kfn = pl.kernel(
        functools.partial(_sc_build_a_kernel, rows_per_worker=rows_per_worker,
                          K=K, E=E),
        mesh=plsc.VectorSubcoreMesh(core_axis_name="c", subcore_axis_name="s"),
        compiler_params=pltpu.CompilerParams(needs_layout_passes=False),
        out_type=jax.ShapeDtypeStruct((R * E,), jnp.float32),
        scratch_types=[
            pltpu.VMEM((n_pairs,), jnp.int32),
            pltpu.VMEM((n_pairs,), jnp.float32),
            pltpu.VMEM((rows_per_worker * E,), jnp.float32),
        ],
    )
    return kfn(idx_flat, w_flat)


def _gelu_exact(x):
    return x * 0.5 * (1.0 + jax.lax.erf(x * 0.7071067811865476))


def _ln(x, g, b, eps=1e-5):
    mu = jnp.mean(x, axis=-1, keepdims=True)
    var = jnp.mean(x * x, axis=-1, keepdims=True) - mu * mu
    return (x - mu) * jax.lax.rsqrt(var + eps) * g + b


def _fused_kernel(a_ref, x_ref, We_ref, be_ref, ge_ref, bbe_ref, Wn_ref,
                  bn_ref, gn_ref, bbn_ref, out_ref, he_s, he2_s, xbf_s, *,
                  NB, Tn):
    nb = pl.program_id(1)

    @pl.when(nb < NB)
    def _phase1():
        x = x_ref[0]
        xbf_s[pl.ds(nb * Tn, Tn), :] = x.astype(jnp.bfloat16)
        a = a_ref[0]
        acc = jax.lax.dot_general(a, x, (((0,), (0,)), ((), ())),
                                  preferred_element_type=jnp.float32)

        @pl.when(nb == 0)
        def _():
            he_s[...] = acc

        @pl.when(nb != 0)
        def _():
            he_s[...] += acc

    @pl.when(nb == NB)
    def _phase2():
        y = jax.lax.dot_general(he_s[...], We_ref[...],
                                (((1,), (0,)), ((), ())),
                                preferred_element_type=jnp.float32)
        y = _gelu_exact(y + be_ref[...])
        he2_s[...] = _ln(y, ge_ref[...], bbe_ref[...])

    @pl.when(nb > NB)
    def _phase3():
        j = nb - NB - 1
        a = a_ref[0]
        xg = jax.lax.dot_general(a, he2_s[...], (((1,), (0,)), ((), ())),
                                 preferred_element_type=jnp.float32)
        y = jax.lax.dot_general(xg.astype(jnp.bfloat16), Wn_ref[...],
                                (((1,), (0,)), ((), ())),
                                preferred_element_type=jnp.float32)
        y = _gelu_exact(y + bn_ref[...])
        y = _ln(y, gn_ref[...], bbn_ref[...])
        out_ref[0] = y + xbf_s[pl.ds(j * Tn, Tn), :].astype(jnp.float32)


def kernel(X, edge_idx, edge_w, We, be, ge, bbe, Wn, bn, gn, bbn):
    B, N, D = X.shape
    K = edge_idx.shape[-1]
    E = 64
    Tn = 1024
    NB = N // Tn
    S = 2 * NB + 1

    idx = edge_idx.astype(jnp.int32)

    a_mat = _sc_build_a(idx.reshape(B * N * K), edge_w.reshape(B * N * K),
                        B * N, K, E).reshape(B, N, E)

    be2 = be.reshape(1, D)
    ge2 = ge.reshape(1, D)
    bbe2 = bbe.reshape(1, D)
    bn2 = bn.reshape(1, D)
    gn2 = gn.reshape(1, D)
    bbn2 = bbn.reshape(1, D)

    def a_map(b, nb):
        j = jnp.where(nb < NB, nb,
                      jnp.clip(nb - NB - 1, 0, NB - 1))
        return (b, j, 0)

    def x_map(b, nb):
        return (b, jnp.minimum(nb, NB - 1), 0)

    def out_map(b, nb):
        return (b, jnp.clip(nb - NB - 1, 0, NB - 1), 0)

    const = lambda b, nb: (0, 0)

    out = pl.pallas_call(
        functools.partial(_fused_kernel, NB=NB, Tn=Tn),
        grid=(B, S),
        in_specs=[
            pl.BlockSpec((1, Tn, E), a_map),
            pl.BlockSpec((1, Tn, D), x_map),
            pl.BlockSpec(We.shape, const),
            pl.BlockSpec((1, D), const),
            pl.BlockSpec((1, D), const),
            pl.BlockSpec((1, D), const),
            pl.BlockSpec(Wn.shape, const),
            pl.BlockSpec((1, D), const),
            pl.BlockSpec((1, D), const),
            pl.BlockSpec((1, D), const),
        ],
        out_specs=pl.BlockSpec((1, Tn, D), out_map),
        out_shape=jax.ShapeDtypeStruct((B, N, D), jnp.float32),
        compiler_params=pltpu.CompilerParams(
            vmem_limit_bytes=100 * 1024 * 1024,
            fuse_transposed_lhs_in_matmul=True),
        scratch_shapes=[
            pltpu.VMEM((E, D), jnp.float32),
            pltpu.VMEM((E, D), jnp.float32),
            pltpu.VMEM((N, D), jnp.bfloat16),
        ],
    )(a_mat, X, We, be2, ge2, bbe2, Wn.astype(jnp.bfloat16), bn2, gn2, bbn2)

    return out
